# baseline (device time: 201410 ns/iter reference)
import jax
import jax.numpy as jnp
from jax import lax
from jax.experimental import pallas as pl
from jax.experimental.pallas import tpu as pltpu

Q_ROWS = 2048
SIZES = (64, 64, 128, 256, 256, 256, 512, 512)
OFFS = tuple(sum(SIZES[:i]) for i in range(len(SIZES)))
CQ = len(SIZES)
DIAG_X_ROWS = 768
DIAG_Y = (5, 6)
DIAG_Z = (7,)
LB_CHUNKS = 8
LB_SLOTS = 4


def kernel(x):
    m_per, n = x.shape

    def body(
        x_ref,
        out_ref,
        lb_vmem,
        lb_h2v_sems,
        lb_v2h_sems,
        sx, rx,
        sxd, rxd,
        s_yd, r_yd,
        s_zd, r_zd,
        s_ydg, r_ydg,
        s_zdg, r_zdg,
    ):
        my_x = lax.axis_index("x")
        my_y = lax.axis_index("y")
        my_z = lax.axis_index("z")
        p_dev = (1 - my_x, my_y, my_z)
        yn_dev = (my_x, 1 - my_y, my_z)
        zn_dev = (my_x, my_y, 1 - my_z)

        mybase = my_x * m_per
        pbase = (1 - my_x) * m_per

        q_me = 2 * my_y + my_z
        q_yn = 2 * (1 - my_y) + my_z
        q_zn = 2 * my_y + (1 - my_z)
        q_diag = 2 * (1 - my_y) + (1 - my_z)

        barrier_sem = pltpu.get_barrier_semaphore()
        for nbr in (p_dev, yn_dev, zn_dev):
            pl.semaphore_signal(
                barrier_sem, inc=1,
                device_id=nbr, device_id_type=pl.DeviceIdType.MESH,
            )
        pl.semaphore_wait(barrier_sem, 3)

        def remote(src, dst, send_sem, recv_sem, dev):
            return pltpu.make_async_remote_copy(
                src_ref=src, dst_ref=dst,
                send_sem=send_sem, recv_sem=recv_sem,
                device_id=dev, device_id_type=pl.DeviceIdType.MESH,
            )

        def out_rows(start, size):
            return out_ref.at[pl.ds(start, size)]

        sends = []

        for c in range(CQ):
            r = remote(
                x_ref.at[pl.ds(q_me * Q_ROWS + OFFS[c], SIZES[c])],
                out_rows(mybase + q_me * Q_ROWS + OFFS[c], SIZES[c]),
                sx.at[c], rx.at[c], p_dev,
            )
            r.start()
            sends.append(r)
        r = remote(
            x_ref.at[pl.ds(q_diag * Q_ROWS, DIAG_X_ROWS)],
            out_rows(mybase + q_diag * Q_ROWS, DIAG_X_ROWS),
            sxd, rxd, p_dev,
        )
        r.start()
        sends.append(r)

        lb_rows = m_per // LB_CHUNKS
        lb_h2v = []
        lb_v2h = []

        def lb_start_h2v(c):
            s = c % LB_SLOTS
            h = pltpu.make_async_copy(
                x_ref.at[pl.ds(c * lb_rows, lb_rows)], lb_vmem.at[s],
                lb_h2v_sems.at[s],
            )
            h.start()
            lb_h2v.append(h)

        def lb_start_v2h(c):
            s = c % LB_SLOTS
            v = pltpu.make_async_copy(
                lb_vmem.at[s],
                out_ref.at[pl.ds(mybase + c * lb_rows, lb_rows)],
                lb_v2h_sems.at[s],
            )
            v.start()
            lb_v2h.append(v)

        def lb_step(c):
            if c < LB_CHUNKS:
                if c >= LB_SLOTS:
                    lb_v2h[c - LB_SLOTS].wait()
                lb_start_h2v(c)
                if c >= 1:
                    lb_h2v[c - 1].wait()
                    lb_start_v2h(c - 1)
            elif c == LB_CHUNKS:
                lb_h2v[LB_CHUNKS - 1].wait()
                lb_start_v2h(LB_CHUNKS - 1)

        def recv(dst, size, recv_sem):
            return remote(
                x_ref.at[pl.ds(0, size)], dst, sx.at[0], recv_sem, p_dev
            )

        for c in range(CQ):
            lb_step(c)
            off = pbase + q_me * Q_ROWS + OFFS[c]
            recv(out_rows(off, SIZES[c]), SIZES[c], rx.at[c]).wait_recv()
            r = remote(
                out_rows(off, SIZES[c]), out_rows(off, SIZES[c]),
                s_yd.at[c], r_yd.at[c], yn_dev,
            )
            r.start()
            sends.append(r)
            r = remote(
                out_rows(off, SIZES[c]), out_rows(off, SIZES[c]),
                s_zd.at[c], r_zd.at[c], zn_dev,
            )
            r.start()
            sends.append(r)
        lb_step(LB_CHUNKS)

        for c in range(CQ):
            recv(
                out_rows(pbase + q_yn * Q_ROWS + OFFS[c], SIZES[c]),
                SIZES[c], r_yd.at[c],
            ).wait_recv()
            if c in DIAG_Z:
                off = pbase + q_yn * Q_ROWS + OFFS[c]
                r = remote(
                    out_rows(off, SIZES[c]), out_rows(off, SIZES[c]),
                    s_zdg, r_zdg, zn_dev,
                )
                r.start()
                sends.append(r)
            recv(
                out_rows(pbase + q_zn * Q_ROWS + OFFS[c], SIZES[c]),
                SIZES[c], r_zd.at[c],
            ).wait_recv()
            if c in DIAG_Y:
                off = pbase + q_zn * Q_ROWS + OFFS[c]
                i = DIAG_Y.index(c)
                r = remote(
                    out_rows(off, SIZES[c]), out_rows(off, SIZES[c]),
                    s_ydg.at[i], r_ydg.at[i], yn_dev,
                )
                r.start()
                sends.append(r)

        recv(
            out_rows(pbase + q_diag * Q_ROWS, DIAG_X_ROWS), DIAG_X_ROWS, rxd
        ).wait_recv()
        for i, c in enumerate(DIAG_Y):
            recv(
                out_rows(pbase + q_diag * Q_ROWS + OFFS[c], SIZES[c]),
                SIZES[c], r_ydg.at[i],
            ).wait_recv()
        for c in DIAG_Z:
            recv(
                out_rows(pbase + q_diag * Q_ROWS + OFFS[c], SIZES[c]),
                SIZES[c], r_zdg,
            ).wait_recv()

        for c in range(max(0, LB_CHUNKS - LB_SLOTS), LB_CHUNKS):
            lb_v2h[c].wait()
        for r in sends:
            r.wait_send()

    lb_rows = m_per // LB_CHUNKS
    return pl.pallas_call(
        body,
        out_shape=jax.ShapeDtypeStruct((2 * m_per, n), x.dtype),
        in_specs=[pl.BlockSpec(memory_space=pl.ANY)],
        out_specs=pl.BlockSpec(memory_space=pl.ANY),
        scratch_shapes=[
            pltpu.VMEM((LB_SLOTS, lb_rows, n), jnp.float32),
            pltpu.SemaphoreType.DMA((LB_SLOTS,)),
            pltpu.SemaphoreType.DMA((LB_SLOTS,)),
            pltpu.SemaphoreType.DMA((CQ,)),
            pltpu.SemaphoreType.DMA((CQ,)),
            pltpu.SemaphoreType.DMA,
            pltpu.SemaphoreType.DMA,
            pltpu.SemaphoreType.DMA((CQ,)),
            pltpu.SemaphoreType.DMA((CQ,)),
            pltpu.SemaphoreType.DMA((CQ,)),
            pltpu.SemaphoreType.DMA((CQ,)),
            pltpu.SemaphoreType.DMA((len(DIAG_Y),)),
            pltpu.SemaphoreType.DMA((len(DIAG_Y),)),
            pltpu.SemaphoreType.DMA,
            pltpu.SemaphoreType.DMA,
        ],
        compiler_params=pltpu.CompilerParams(collective_id=0),
    )(x)


# device time: 190725 ns/iter; 1.0560x vs baseline; 1.0560x over previous
import jax
import jax.numpy as jnp
from jax import lax
from jax.experimental import pallas as pl
from jax.experimental.pallas import tpu as pltpu

Q_ROWS = 2048
SIZES = (256, 256, 256, 256, 256, 256, 256, 256)
OFFS = tuple(sum(SIZES[:i]) for i in range(len(SIZES)))
CQ = len(SIZES)
DIAG_X_ROWS = 768
DIAG_Y = (3, 4, 5)
DIAG_Z = (6, 7)
LB_CHUNKS = 8
LB_SLOTS = 4


def kernel(x):
    m_per, n = x.shape

    def body(
        x_ref,
        out_ref,
        lb_vmem,
        lb_h2v_sems,
        lb_v2h_sems,
        sx, rx,
        sxd, rxd,
        s_yd, r_yd,
        s_zd, r_zd,
        s_ydg, r_ydg,
        s_zdg, r_zdg,
    ):
        my_x = lax.axis_index("x")
        my_y = lax.axis_index("y")
        my_z = lax.axis_index("z")
        p_dev = (1 - my_x, my_y, my_z)
        yn_dev = (my_x, 1 - my_y, my_z)
        zn_dev = (my_x, my_y, 1 - my_z)

        mybase = my_x * m_per
        pbase = (1 - my_x) * m_per

        q_me = 2 * my_y + my_z
        q_yn = 2 * (1 - my_y) + my_z
        q_zn = 2 * my_y + (1 - my_z)
        q_diag = 2 * (1 - my_y) + (1 - my_z)

        barrier_sem = pltpu.get_barrier_semaphore()
        for nbr in (p_dev, yn_dev, zn_dev):
            pl.semaphore_signal(
                barrier_sem, inc=1,
                device_id=nbr, device_id_type=pl.DeviceIdType.MESH,
            )
        pl.semaphore_wait(barrier_sem, 3)

        def remote(src, dst, send_sem, recv_sem, dev):
            return pltpu.make_async_remote_copy(
                src_ref=src, dst_ref=dst,
                send_sem=send_sem, recv_sem=recv_sem,
                device_id=dev, device_id_type=pl.DeviceIdType.MESH,
            )

        def out_rows(start, size):
            return out_ref.at[pl.ds(start, size)]

        sends = []

        for c in range(CQ):
            r = remote(
                x_ref.at[pl.ds(q_me * Q_ROWS + OFFS[c], SIZES[c])],
                out_rows(mybase + q_me * Q_ROWS + OFFS[c], SIZES[c]),
                sx.at[c], rx.at[c], p_dev,
            )
            r.start()
            sends.append(r)
        r = remote(
            x_ref.at[pl.ds(q_diag * Q_ROWS, DIAG_X_ROWS)],
            out_rows(mybase + q_diag * Q_ROWS, DIAG_X_ROWS),
            sxd, rxd, p_dev,
        )
        r.start()
        sends.append(r)

        lb_rows = m_per // LB_CHUNKS
        lb_h2v = []
        lb_v2h = []

        def lb_start_h2v(c):
            s = c % LB_SLOTS
            h = pltpu.make_async_copy(
                x_ref.at[pl.ds(c * lb_rows, lb_rows)], lb_vmem.at[s],
                lb_h2v_sems.at[s],
            )
            h.start()
            lb_h2v.append(h)

        def lb_start_v2h(c):
            s = c % LB_SLOTS
            v = pltpu.make_async_copy(
                lb_vmem.at[s],
                out_ref.at[pl.ds(mybase + c * lb_rows, lb_rows)],
                lb_v2h_sems.at[s],
            )
            v.start()
            lb_v2h.append(v)

        def lb_step(c):
            if c < LB_CHUNKS:
                if c >= LB_SLOTS:
                    lb_v2h[c - LB_SLOTS].wait()
                lb_start_h2v(c)
                if c >= 1:
                    lb_h2v[c - 1].wait()
                    lb_start_v2h(c - 1)
            elif c == LB_CHUNKS:
                lb_h2v[LB_CHUNKS - 1].wait()
                lb_start_v2h(LB_CHUNKS - 1)

        def recv(dst, size, recv_sem):
            return remote(
                x_ref.at[pl.ds(0, size)], dst, sx.at[0], recv_sem, p_dev
            )

        for c in range(CQ):
            lb_step(c)
            off = pbase + q_me * Q_ROWS + OFFS[c]
            recv(out_rows(off, SIZES[c]), SIZES[c], rx.at[c]).wait_recv()
            r = remote(
                out_rows(off, SIZES[c]), out_rows(off, SIZES[c]),
                s_yd.at[c], r_yd.at[c], yn_dev,
            )
            r.start()
            sends.append(r)
            r = remote(
                out_rows(off, SIZES[c]), out_rows(off, SIZES[c]),
                s_zd.at[c], r_zd.at[c], zn_dev,
            )
            r.start()
            sends.append(r)
        lb_step(LB_CHUNKS)

        for c in range(CQ):
            recv(
                out_rows(pbase + q_yn * Q_ROWS + OFFS[c], SIZES[c]),
                SIZES[c], r_yd.at[c],
            ).wait_recv()
            if c in DIAG_Z:
                off = pbase + q_yn * Q_ROWS + OFFS[c]
                i = DIAG_Z.index(c)
                r = remote(
                    out_rows(off, SIZES[c]), out_rows(off, SIZES[c]),
                    s_zdg.at[i], r_zdg.at[i], zn_dev,
                )
                r.start()
                sends.append(r)
            recv(
                out_rows(pbase + q_zn * Q_ROWS + OFFS[c], SIZES[c]),
                SIZES[c], r_zd.at[c],
            ).wait_recv()
            if c in DIAG_Y:
                off = pbase + q_zn * Q_ROWS + OFFS[c]
                i = DIAG_Y.index(c)
                r = remote(
                    out_rows(off, SIZES[c]), out_rows(off, SIZES[c]),
                    s_ydg.at[i], r_ydg.at[i], yn_dev,
                )
                r.start()
                sends.append(r)

        recv(
            out_rows(pbase + q_diag * Q_ROWS, DIAG_X_ROWS), DIAG_X_ROWS, rxd
        ).wait_recv()
        for i, c in enumerate(DIAG_Y):
            recv(
                out_rows(pbase + q_diag * Q_ROWS + OFFS[c], SIZES[c]),
                SIZES[c], r_ydg.at[i],
            ).wait_recv()
        for i, c in enumerate(DIAG_Z):
            recv(
                out_rows(pbase + q_diag * Q_ROWS + OFFS[c], SIZES[c]),
                SIZES[c], r_zdg.at[i],
            ).wait_recv()

        for c in range(max(0, LB_CHUNKS - LB_SLOTS), LB_CHUNKS):
            lb_v2h[c].wait()
        for r in sends:
            r.wait_send()

    lb_rows = m_per // LB_CHUNKS
    return pl.pallas_call(
        body,
        out_shape=jax.ShapeDtypeStruct((2 * m_per, n), x.dtype),
        in_specs=[pl.BlockSpec(memory_space=pl.ANY)],
        out_specs=pl.BlockSpec(memory_space=pl.ANY),
        scratch_shapes=[
            pltpu.VMEM((LB_SLOTS, lb_rows, n), jnp.float32),
            pltpu.SemaphoreType.DMA((LB_SLOTS,)),
            pltpu.SemaphoreType.DMA((LB_SLOTS,)),
            pltpu.SemaphoreType.DMA((CQ,)),
            pltpu.SemaphoreType.DMA((CQ,)),
            pltpu.SemaphoreType.DMA,
            pltpu.SemaphoreType.DMA,
            pltpu.SemaphoreType.DMA((CQ,)),
            pltpu.SemaphoreType.DMA((CQ,)),
            pltpu.SemaphoreType.DMA((CQ,)),
            pltpu.SemaphoreType.DMA((CQ,)),
            pltpu.SemaphoreType.DMA((len(DIAG_Y),)),
            pltpu.SemaphoreType.DMA((len(DIAG_Y),)),
            pltpu.SemaphoreType.DMA((len(DIAG_Z),)),
            pltpu.SemaphoreType.DMA((len(DIAG_Z),)),
        ],
        compiler_params=pltpu.CompilerParams(collective_id=0),
    )(x)


# device time: 189762 ns/iter; 1.0614x vs baseline; 1.0051x over previous
import jax
import jax.numpy as jnp
from jax import lax
from jax.experimental import pallas as pl
from jax.experimental.pallas import tpu as pltpu

Q_ROWS = 2048
SIZES = (256,) * 8
OFFS = tuple(sum(SIZES[:i]) for i in range(len(SIZES)))
CQ = len(SIZES)
DIAG_X_ROWS = 768
DIAG_Y = (3, 4, 5)
DIAG_Z = (6, 7)
LB_CHUNKS = 8
LB_SLOTS = 4


def kernel(x):
    m_per, n = x.shape

    def body(
        x_ref,
        out_ref,
        lb_vmem,
        lb_h2v_sems,
        lb_v2h_sems,
        sx, rx,
        sxd, rxd,
        s_yd, r_yd,
        s_zd, r_zd,
        s_ydg, r_ydg,
        s_zdg, r_zdg,
    ):
        my_x = lax.axis_index("x")
        my_y = lax.axis_index("y")
        my_z = lax.axis_index("z")
        p_dev = (1 - my_x, my_y, my_z)
        yn_dev = (my_x, 1 - my_y, my_z)
        zn_dev = (my_x, my_y, 1 - my_z)

        mybase = my_x * m_per
        pbase = (1 - my_x) * m_per

        q_me = 2 * my_y + my_z
        q_yn = 2 * (1 - my_y) + my_z
        q_zn = 2 * my_y + (1 - my_z)
        q_diag = 2 * (1 - my_y) + (1 - my_z)

        barrier_sem = pltpu.get_barrier_semaphore()
        for nbr in (p_dev, yn_dev, zn_dev):
            pl.semaphore_signal(
                barrier_sem, inc=1,
                device_id=nbr, device_id_type=pl.DeviceIdType.MESH,
            )
        pl.semaphore_wait(barrier_sem, 3)

        def remote(src, dst, send_sem, recv_sem, dev):
            return pltpu.make_async_remote_copy(
                src_ref=src, dst_ref=dst,
                send_sem=send_sem, recv_sem=recv_sem,
                device_id=dev, device_id_type=pl.DeviceIdType.MESH,
            )

        def out_rows(start, size):
            return out_ref.at[pl.ds(start, size)]

        sends = []

        for c in range(CQ):
            r = remote(
                x_ref.at[pl.ds(q_me * Q_ROWS + OFFS[c], SIZES[c])],
                out_rows(mybase + q_me * Q_ROWS + OFFS[c], SIZES[c]),
                sx.at[c], rx.at[c], p_dev,
            )
            r.start()
            sends.append(r)
        r = remote(
            x_ref.at[pl.ds(q_diag * Q_ROWS, DIAG_X_ROWS)],
            out_rows(mybase + q_diag * Q_ROWS, DIAG_X_ROWS),
            sxd, rxd, p_dev,
        )
        r.start()
        sends.append(r)

        lb_rows = m_per // LB_CHUNKS
        lb_h2v = []
        lb_v2h = []

        def lb_start_h2v(c):
            s = c % LB_SLOTS
            h = pltpu.make_async_copy(
                x_ref.at[pl.ds(c * lb_rows, lb_rows)], lb_vmem.at[s],
                lb_h2v_sems.at[s],
            )
            h.start()
            lb_h2v.append(h)

        def lb_start_v2h(c):
            s = c % LB_SLOTS
            v = pltpu.make_async_copy(
                lb_vmem.at[s],
                out_ref.at[pl.ds(mybase + c * lb_rows, lb_rows)],
                lb_v2h_sems.at[s],
            )
            v.start()
            lb_v2h.append(v)

        def lb_step(c):
            if c < LB_CHUNKS:
                if c >= LB_SLOTS:
                    lb_v2h[c - LB_SLOTS].wait()
                lb_start_h2v(c)
                if c >= 1:
                    lb_h2v[c - 1].wait()
                    lb_start_v2h(c - 1)
            elif c == LB_CHUNKS:
                lb_h2v[LB_CHUNKS - 1].wait()
                lb_start_v2h(LB_CHUNKS - 1)

        def recv(dst, size, recv_sem):
            return remote(
                x_ref.at[pl.ds(0, size)], dst, sx.at[0], recv_sem, p_dev
            )

        for c in range(CQ):
            lb_step(c)
            off = pbase + q_me * Q_ROWS + OFFS[c]
            recv(out_rows(off, SIZES[c]), SIZES[c], rx.at[c]).wait_recv()
            r = remote(
                out_rows(off, SIZES[c]), out_rows(off, SIZES[c]),
                s_yd.at[c], r_yd.at[c], yn_dev,
            )
            r.start()
            sends.append(r)
            r = remote(
                out_rows(off, SIZES[c]), out_rows(off, SIZES[c]),
                s_zd.at[c], r_zd.at[c], zn_dev,
            )
            r.start()
            sends.append(r)
        lb_step(LB_CHUNKS)

        for c in range(CQ):
            recv(
                out_rows(pbase + q_yn * Q_ROWS + OFFS[c], SIZES[c]),
                SIZES[c], r_yd.at[c],
            ).wait_recv()
            if c in DIAG_Z:
                off = pbase + q_yn * Q_ROWS + OFFS[c]
                i = DIAG_Z.index(c)
                r = remote(
                    out_rows(off, SIZES[c]), out_rows(off, SIZES[c]),
                    s_zdg.at[i], r_zdg.at[i], zn_dev,
                )
                r.start()
                sends.append(r)
            recv(
                out_rows(pbase + q_zn * Q_ROWS + OFFS[c], SIZES[c]),
                SIZES[c], r_zd.at[c],
            ).wait_recv()
            if c in DIAG_Y:
                off = pbase + q_zn * Q_ROWS + OFFS[c]
                i = DIAG_Y.index(c)
                r = remote(
                    out_rows(off, SIZES[c]), out_rows(off, SIZES[c]),
                    s_ydg.at[i], r_ydg.at[i], yn_dev,
                )
                r.start()
                sends.append(r)

        recv(
            out_rows(pbase + q_diag * Q_ROWS, DIAG_X_ROWS), DIAG_X_ROWS, rxd
        ).wait_recv()
        for i, c in enumerate(DIAG_Y):
            recv(
                out_rows(pbase + q_diag * Q_ROWS + OFFS[c], SIZES[c]),
                SIZES[c], r_ydg.at[i],
            ).wait_recv()
        for i, c in enumerate(DIAG_Z):
            recv(
                out_rows(pbase + q_diag * Q_ROWS + OFFS[c], SIZES[c]),
                SIZES[c], r_zdg.at[i],
            ).wait_recv()

        for c in range(max(0, LB_CHUNKS - LB_SLOTS), LB_CHUNKS):
            lb_v2h[c].wait()
        for r in sends:
            r.wait_send()

    lb_rows = m_per // LB_CHUNKS
    return pl.pallas_call(
        body,
        out_shape=jax.ShapeDtypeStruct((2 * m_per, n), x.dtype),
        in_specs=[pl.BlockSpec(memory_space=pl.ANY)],
        out_specs=pl.BlockSpec(memory_space=pl.ANY),
        scratch_shapes=[
            pltpu.VMEM((LB_SLOTS, lb_rows, n), jnp.float32),
            pltpu.SemaphoreType.DMA((LB_SLOTS,)),
            pltpu.SemaphoreType.DMA((LB_SLOTS,)),
            pltpu.SemaphoreType.DMA((CQ,)),
            pltpu.SemaphoreType.DMA((CQ,)),
            pltpu.SemaphoreType.DMA,
            pltpu.SemaphoreType.DMA,
            pltpu.SemaphoreType.DMA((CQ,)),
            pltpu.SemaphoreType.DMA((CQ,)),
            pltpu.SemaphoreType.DMA((CQ,)),
            pltpu.SemaphoreType.DMA((CQ,)),
            pltpu.SemaphoreType.DMA((len(DIAG_Y),)),
            pltpu.SemaphoreType.DMA((len(DIAG_Y),)),
            pltpu.SemaphoreType.DMA((len(DIAG_Z),)),
            pltpu.SemaphoreType.DMA((len(DIAG_Z),)),
        ],
        compiler_params=pltpu.CompilerParams(collective_id=0),
    )(x)
